# baseline (speedup 1.0000x reference)
import jax
import jax.numpy as jnp
from jax.experimental import pallas as pl
from jax.experimental.pallas import tpu as pltpu

_ROWS = 1024
_LBL = 512
_BLK = 128
_GRID = _ROWS // _BLK


def _loss_body(p_ref, s_ref, a_ref, out_ref, cand_ref, acc_ref):
    i = pl.program_id(0)

    @pl.when(i == 0)
    def _init():
        inv_l = jnp.float32(1.0 / _LBL)
        cand_ref[...] = (
            jnp.dot(p_ref[...], a_ref[...], preferred_element_type=jnp.float32)
            * inv_l
        )
        acc_ref[0] = jnp.float32(0.0)
        acc_ref[1] = jnp.float32(0.0)

    s = s_ref[...]
    p_blk = p_ref[pl.ds(i * _BLK, _BLK), :]
    diff = p_blk - jnp.dot(s, cand_ref[...], preferred_element_type=jnp.float32)
    sq = jnp.sum(diff * diff, axis=1)
    norms = jnp.sqrt(sq)
    mask = jnp.sum(s, axis=1) != 0
    acc_ref[0] += jnp.sum(jnp.where(mask, norms, jnp.float32(0.0)))
    acc_ref[1] += jnp.sum(mask.astype(jnp.float32))

    @pl.when(i == _GRID - 1)
    def _emit():
        out_ref[...] = jnp.reshape(acc_ref[0] / acc_ref[1], (1, 1))


def kernel(predicts, similarities, adjList):
    out = pl.pallas_call(
        _loss_body,
        grid=(_GRID,),
        in_specs=[
            pl.BlockSpec((_ROWS, _LBL), lambda i: (0, 0)),
            pl.BlockSpec((_BLK, _ROWS), lambda i: (i, 0)),
            pl.BlockSpec((_LBL, _LBL), lambda i: (0, 0)),
        ],
        out_specs=pl.BlockSpec((1, 1), lambda i: (0, 0)),
        out_shape=jax.ShapeDtypeStruct((1, 1), jnp.float32),
        scratch_shapes=[
            pltpu.VMEM((_ROWS, _LBL), jnp.float32),
            pltpu.SMEM((2,), jnp.float32),
        ],
    )(predicts, similarities, adjList)
    return out[0, 0]
